# Initial kernel scaffold; baseline (speedup 1.0000x reference)
import jax
import jax.numpy as jnp
from jax.experimental import pallas as pl


def kernel(noise, order):
    raise NotImplementedError("write your pallas kernel here")


# speedup vs baseline: 1.0447x; 1.0447x over previous
import functools

import jax
import jax.numpy as jnp
from jax import lax
from jax.experimental import pallas as pl
from jax.experimental.pallas import tpu as pltpu
from jax.experimental.pallas import tpu_sc as plsc

_SC_INFO = plsc.get_sparse_core_info()
_NC = _SC_INFO.num_cores
_NS = _SC_INFO.num_subcores
_NW = _NC * _NS


def _idx_body(n, ncols, noise_ref, out_ref):
    x = noise_ref[...]
    unif = 0.5 * (1.0 + lax.erf(x / jnp.sqrt(jnp.asarray(2.0, x.dtype))))
    idx = jnp.floor(unif * n).astype(jnp.int32)
    idx = jnp.minimum(idx, n - 1)
    col = lax.broadcasted_iota(jnp.int32, x.shape, 1)
    out_ref[...] = idx * ncols + col


def _flat_indices(noise, n, ncols):
    b = noise.shape[0]
    block_rows = 2048
    grid = (b // block_rows,)
    return pl.pallas_call(
        functools.partial(_idx_body, n, ncols),
        grid=grid,
        in_specs=[pl.BlockSpec((block_rows, ncols), lambda i: (i, 0))],
        out_specs=pl.BlockSpec((block_rows, ncols), lambda i: (i, 0)),
        out_shape=jax.ShapeDtypeStruct((b, ncols), jnp.int32),
    )(noise)


def _make_sc_gather(total, chunk):
    per_w = total // _NW
    n_chunks = per_w // chunk
    mesh = plsc.VectorSubcoreMesh(core_axis_name="c", subcore_axis_name="s")

    @functools.partial(
        pl.kernel,
        mesh=mesh,
        out_type=jax.ShapeDtypeStruct((total,), jnp.float32),
        scratch_types=[
            pltpu.VMEM((chunk,), jnp.int32),
            pltpu.VMEM((chunk,), jnp.float32),
            pltpu.SemaphoreType.DMA,
        ],
    )
    def sc_gather(order_hbm, idx_hbm, out_hbm, idx_v, rows_v, sem):
        wid = lax.axis_index("s") * _NC + lax.axis_index("c")
        base = wid * per_w
        for k in range(n_chunks):
            off = base + k * chunk
            pltpu.sync_copy(idx_hbm.at[pl.ds(off, chunk)], idx_v)
            pltpu.async_copy(order_hbm.at[idx_v], rows_v, sem).wait()
            pltpu.sync_copy(rows_v, out_hbm.at[pl.ds(off, chunk)])

    return sc_gather


def kernel(noise, order):
    n, ncols = order.shape
    b = noise.shape[0]
    flat_idx = _flat_indices(noise, n, ncols).reshape(-1)
    order_flat = order.reshape(-1)
    total = b * ncols
    out_flat = _make_sc_gather(total, 16384)(order_flat, flat_idx)
    return out_flat.reshape(b, ncols)


# speedup vs baseline: 1.0854x; 1.0389x over previous
import functools

import jax
import jax.numpy as jnp
from jax import lax
from jax.experimental import pallas as pl
from jax.experimental.pallas import tpu as pltpu
from jax.experimental.pallas import tpu_sc as plsc

_SC_INFO = plsc.get_sparse_core_info()
_NC = _SC_INFO.num_cores
_NS = _SC_INFO.num_subcores
_NW = _NC * _NS


def _idx_body(n, ncols, noise_ref, out_ref):
    x = noise_ref[...]
    unif = 0.5 * (1.0 + lax.erf(x / jnp.sqrt(jnp.asarray(2.0, x.dtype))))
    idx = jnp.floor(unif * n).astype(jnp.int32)
    idx = jnp.minimum(idx, n - 1)
    col = lax.broadcasted_iota(jnp.int32, x.shape, 1)
    out_ref[...] = idx * ncols + col


def _flat_indices(noise, n, ncols):
    b = noise.shape[0]
    block_rows = 2048
    grid = (b // block_rows,)
    return pl.pallas_call(
        functools.partial(_idx_body, n, ncols),
        grid=grid,
        in_specs=[pl.BlockSpec((block_rows, ncols), lambda i: (i, 0))],
        out_specs=pl.BlockSpec((block_rows, ncols), lambda i: (i, 0)),
        out_shape=jax.ShapeDtypeStruct((b, ncols), jnp.int32),
    )(noise)


def _make_sc_gather(total, chunk):
    per_w = total // _NW
    n_chunks = per_w // chunk
    mesh = plsc.VectorSubcoreMesh(core_axis_name="c", subcore_axis_name="s")

    @functools.partial(
        pl.kernel,
        mesh=mesh,
        out_type=jax.ShapeDtypeStruct((total,), jnp.float32),
        scratch_types=[
            pltpu.VMEM((chunk,), jnp.int32),
            pltpu.VMEM((chunk,), jnp.int32),
            pltpu.VMEM((chunk,), jnp.float32),
            pltpu.VMEM((chunk,), jnp.float32),
            pltpu.SemaphoreType.DMA,
            pltpu.SemaphoreType.DMA,
            pltpu.SemaphoreType.DMA,
            pltpu.SemaphoreType.DMA,
        ],
    )
    def sc_gather(order_hbm, idx_hbm, out_hbm,
                  idx_v0, idx_v1, rows_v0, rows_v1, g0, g1, o0, o1):
        wid = lax.axis_index("s") * _NC + lax.axis_index("c")
        base = wid * per_w
        idx_bufs = (idx_v0, idx_v1)
        row_bufs = (rows_v0, rows_v1)
        gsems = (g0, g1)
        osems = (o0, o1)
        gathers = [None, None]
        stores = [None, None]
        for k in range(n_chunks):
            s = k % 2
            off = base + k * chunk
            if stores[s] is not None:
                stores[s].wait()
            pltpu.sync_copy(idx_hbm.at[pl.ds(off, chunk)], idx_bufs[s])
            gathers[s] = pltpu.async_copy(
                order_hbm.at[idx_bufs[s]], row_bufs[s], gsems[s])
            p = 1 - s
            if gathers[p] is not None:
                gathers[p].wait()
                poff = base + (k - 1) * chunk
                stores[p] = pltpu.async_copy(
                    row_bufs[p], out_hbm.at[pl.ds(poff, chunk)], osems[p])
        s_last = (n_chunks - 1) % 2
        gathers[s_last].wait()
        last_off = base + (n_chunks - 1) * chunk
        stores[s_last] = pltpu.async_copy(
            row_bufs[s_last], out_hbm.at[pl.ds(last_off, chunk)], osems[s_last])
        stores[0].wait()
        stores[1].wait()

    return sc_gather


def kernel(noise, order):
    n, ncols = order.shape
    b = noise.shape[0]
    flat_idx = _flat_indices(noise, n, ncols).reshape(-1)
    order_flat = order.reshape(-1)
    total = b * ncols
    out_flat = _make_sc_gather(total, 16384)(order_flat, flat_idx)
    return out_flat.reshape(b, ncols)
